# baseline (speedup 1.0000x reference)
import functools
import math

import jax
import jax.numpy as jnp
from jax import lax
from jax.experimental import pallas as pl
from jax.experimental.pallas import tpu as pltpu
from jax.experimental.pallas import tpu_sc as plsc

_NC = 2
_NS = 16
_NW = _NC * _NS


def _positional_encoding(embedding_dim, max_len=10000):
    position = jnp.arange(0, max_len, dtype=jnp.float32)[:, None]
    half = embedding_dim // 2
    div_term = jnp.exp(
        jnp.arange(0, half, dtype=jnp.float32) * -(math.log(10000.0) / (half - 1)))
    return jnp.concatenate(
        [jnp.sin(position * div_term), jnp.cos(position * div_term)], axis=1)


@functools.lru_cache(maxsize=None)
def _make_sc_kernel(n_rows, L, D, scale):
    rows_per_worker = n_rows // _NW
    mesh = plsc.VectorSubcoreMesh(core_axis_name="c", subcore_axis_name="s")
    n_a = 128
    n_b = L - n_a

    @functools.partial(
        pl.kernel,
        out_type=jax.ShapeDtypeStruct((n_rows * L, D), jnp.float32),
        mesh=mesh,
        scratch_types=[
            pltpu.VMEM((L, D), jnp.float32),
            pltpu.VMEM((n_a,), jnp.int32),
            pltpu.VMEM((n_b,), jnp.int32),
            pltpu.VMEM((L, D), jnp.float32),
            pltpu.SemaphoreType.DMA,
        ],
        compiler_params=pltpu.CompilerParams(use_tc_tiling_on_sc=False),
    )
    def sc_fn(table_hbm, x_hbm, pe_hbm, out_hbm, pe_v, idx_a, idx_b, rows_v, sem):
        wid = lax.axis_index("s") * _NC + lax.axis_index("c")
        pltpu.sync_copy(pe_hbm, pe_v)

        def row_body(j, carry):
            base = (wid * rows_per_worker + j) * L
            pltpu.sync_copy(x_hbm.at[pl.ds(base, n_a)], idx_a)
            pltpu.sync_copy(x_hbm.at[pl.ds(base + n_a, n_b)], idx_b)
            cp1 = pltpu.async_copy(table_hbm.at[idx_a],
                                   rows_v.at[pl.ds(0, n_a)], sem)
            cp2 = pltpu.async_copy(table_hbm.at[idx_b],
                                   rows_v.at[pl.ds(n_a, n_b)], sem)
            cp1.wait()
            cp2.wait()

            def comp(l, c):
                for d0 in range(D // 16):
                    sl = pl.ds(d0 * 16, 16)
                    rows_v[l, sl] = rows_v[l, sl] * scale + pe_v[l, sl]
                return c

            lax.fori_loop(0, L, comp, 0)
            pltpu.sync_copy(rows_v, out_hbm.at[pl.ds(base, L)])
            return carry

        lax.fori_loop(0, rows_per_worker, row_body, 0)

    return sc_fn


def kernel(x, timestep, table):
    n, L = x.shape
    D = table.shape[1]
    scale = float(D ** 0.5)
    pe = _positional_encoding(D)
    pe_slice = lax.dynamic_slice_in_dim(pe, timestep, L, axis=0)
    x_flat = x.reshape(-1)
    sc_fn = _make_sc_kernel(n, L, D, scale)
    out = sc_fn(table, x_flat, pe_slice)
    return out.reshape(n, L, D)
